# baseline (speedup 1.0000x reference)
import functools

import jax
import jax.numpy as jnp
from jax import lax
from jax.experimental import pallas as pl
from jax.experimental.pallas import tpu as pltpu
from jax.experimental.pallas import tpu_sc as plsc

N = 50000
E = 800000
G = 1024
H = 128
N_PAD = 50176
NB5 = N_PAD // 512
NCH = E // 1280
F32 = jnp.float32

@functools.lru_cache(maxsize=1)
def _sc_mesh():
    return plsc.VectorSubcoreMesh(
        core_axis_name="c", subcore_axis_name="s", num_cores=2,
        num_subcores=16)


def _relu(x):
    return jnp.maximum(x, 0.0)


def _dot(a, b):
    return jnp.dot(a, b, preferred_element_type=F32)


def _k1_body(nf, wn, bn, wl, bl_, wr, br_, wu1, wu2, nn1_o, a_o, b_o):
    x = nf[...]
    nn1_o[...] = _relu(_dot(x, wn[...]) + bn[...])
    a_o[...] = _dot(_relu(_dot(x, wl[...]) + bl_[...]), wu1[...])
    b_o[...] = _dot(_relu(_dot(x, wr[...]) + br_[...]), wu2[...])


def _k1(nfp, wn, bn, wl, bl_, wr, br_, wu1, wu2):
    bs = 512
    row = pl.BlockSpec((bs, 32), lambda i: (i, 0))
    w32 = pl.BlockSpec((32, H), lambda i: (0, 0))
    wHH = pl.BlockSpec((H, H), lambda i: (0, 0))
    bia = pl.BlockSpec((1, H), lambda i: (0, 0))
    out = pl.BlockSpec((bs, H), lambda i: (i, 0))
    return pl.pallas_call(
        _k1_body,
        grid=(N_PAD // bs,),
        in_specs=[row, w32, bia, w32, bia, w32, bia, wHH, wHH],
        out_specs=[out, out, out],
        out_shape=[jax.ShapeDtypeStruct((N_PAD, H), F32)] * 3,
    )(nfp, wn, bn, wl, bl_, wr, br_, wu1, wu2)


NCH2 = E // 128


def _k2_body(a_hbm, b_hbm, src_hbm, dst_hbm, out_hbm, si, di, buf,
             semi, semA, semB, semS):
    c = lax.axis_index("c")
    s = lax.axis_index("s")
    w = s * 2 + c

    def k_of(j):
        return w + 32 * j

    def idx_load(j, b):
        k = k_of(j)

        @pl.when(k < NCH2)
        def _():
            pltpu.async_copy(src_hbm.at[pl.ds(k * 128, 128)], si.at[b], semi)
            pltpu.async_copy(dst_hbm.at[pl.ds(k * 128, 128)], di.at[b], semi)

    def idx_wait(b):
        pltpu.make_async_copy(src_hbm.at[pl.ds(0, 128)], si.at[b], semi).wait()
        pltpu.make_async_copy(dst_hbm.at[pl.ds(0, 128)], di.at[b], semi).wait()

    idx_load(0, 0)
    idx_load(1, 1)
    idx_wait(0)
    pltpu.async_copy(a_hbm.at[si.at[0]], buf.at[0], semA)

    def pair(j2, carry):
        for b in (0, 1):
            j = 2 * j2 + b
            k = k_of(j)
            kn = k_of(j + 1)
            bn = 1 - b

            @pl.when(k < NCH2)
            def _():
                pltpu.make_async_copy(a_hbm.at[si.at[b]], buf.at[b],
                                      semA).wait()
                pltpu.async_copy(b_hbm.at[di.at[b]], buf.at[b], semB,
                                 add=True)

            @pl.when(kn < NCH2)
            def _():
                @pl.when(j >= 1)
                def _():
                    pltpu.make_async_copy(
                        buf.at[bn], out_hbm.at[pl.ds(0, 128)], semS).wait()
                idx_wait(bn)
                pltpu.async_copy(a_hbm.at[si.at[bn]], buf.at[bn], semA)

            @pl.when(k < NCH2)
            def _():
                pltpu.make_async_copy(b_hbm.at[di.at[b]], buf.at[b],
                                      semB).wait()
                pltpu.async_copy(buf.at[b], out_hbm.at[pl.ds(k * 128, 128)],
                                 semS)
            idx_load(j + 2, b)
        return carry

    lax.fori_loop(0, (NCH2 // 32 + 2) // 2, pair, 0)
    pltpu.make_async_copy(buf.at[0], out_hbm.at[pl.ds(0, 128)], semS).wait()
    pltpu.make_async_copy(buf.at[1], out_hbm.at[pl.ds(0, 128)], semS).wait()


def _k2(a_t, b_t, src, dst):
    f = pl.kernel(
        _k2_body,
        out_type=jax.ShapeDtypeStruct((E, H), F32),
        mesh=_sc_mesh(),
        compiler_params=pltpu.CompilerParams(use_tc_tiling_on_sc=False),
        cost_estimate=pl.CostEstimate(
            flops=0, transcendentals=0, bytes_accessed=E * H * 4 * 3),
        scratch_types=[
            pltpu.VMEM((2, 128), jnp.int32),
            pltpu.VMEM((2, 128), jnp.int32),
            pltpu.VMEM((2, 128, H), F32),
            pltpu.SemaphoreType.DMA,
            pltpu.SemaphoreType.DMA,
            pltpu.SemaphoreType.DMA,
            pltpu.SemaphoreType.DMA,
        ],
    )
    return f(a_t, b_t, src, dst)


def _k3_body(ef, pre, wzn, bzn, wee, bee, wu3, bue, w4, b4, zy_o):
    bf = jnp.bfloat16
    e = ef[...]
    z = _relu(_dot(e, wzn[...]) + bzn[...])
    e2e = _relu(_dot(e, wee[...]) + bee[...])
    cc = _dot(e2e.astype(bf), wu3[...].astype(bf)) + bue[...]
    ne = _relu(pre[...] + cc)
    y = _relu(_dot(ne.astype(bf), w4[...].astype(bf)) + b4[...])
    zy_o[0, :, :] = z
    zy_o[1, :, :] = y


def _k3(ef, pre, wzn, bzn, wee, bee, wu3, bue, w4, b4):
    bs = 640
    return pl.pallas_call(
        _k3_body,
        grid=(E // bs,),
        in_specs=[
            pl.BlockSpec((bs, 6), lambda i: (i, 0)),
            pl.BlockSpec((bs, H), lambda i: (i, 0)),
            pl.BlockSpec((6, H), lambda i: (0, 0)),
            pl.BlockSpec((1, H), lambda i: (0, 0)),
            pl.BlockSpec((6, H), lambda i: (0, 0)),
            pl.BlockSpec((1, H), lambda i: (0, 0)),
            pl.BlockSpec((H, H), lambda i: (0, 0)),
            pl.BlockSpec((1, H), lambda i: (0, 0)),
            pl.BlockSpec((H, H), lambda i: (0, 0)),
            pl.BlockSpec((1, H), lambda i: (0, 0)),
        ],
        out_specs=pl.BlockSpec((2, bs, H), lambda i: (0, i, 0)),
        out_shape=jax.ShapeDtypeStruct((2, E, H), F32),
    )(ef, pre, wzn, bzn, wee, bee, wu3, bue, w4, b4)


NCH4 = E // 256


def _k4_body(zy_hbm, d2_hbm, out_hbm, idx_v, data_v, zero_v, table, semL):
    c = lax.axis_index("c")
    s = lax.axis_index("s")
    zf = jnp.zeros((16,), F32)
    for r in range(64):
        for q in range(2):
            zero_v[r, pl.ds(q * 16, 16)] = zf

    rows_per_tile = N_PAD // 16
    row0 = s * rows_per_tile

    def k_of(j):
        return s + 16 * j

    for grp in range(4):
        def load(j, b):
            k = k_of(j)

            @pl.when(k < NCH4)
            def _():
                pltpu.async_copy(d2_hbm.at[pl.ds(k * 2, 2)], idx_v.at[b],
                                 semL)
                pltpu.async_copy(
                    zy_hbm.at[c, pl.ds(k * 256, 256), pl.ds(grp * 32, 32)],
                    data_v.at[b], semL)

        def load_wait(b):
            pltpu.make_async_copy(d2_hbm.at[pl.ds(0, 2)], idx_v.at[b],
                                  semL).wait()
            pltpu.make_async_copy(
                zy_hbm.at[c, pl.ds(0, 256), pl.ds(grp * 32, 32)],
                data_v.at[b], semL).wait()

        load(0, 0)

        def zstep(i, carry):
            pltpu.sync_copy(zero_v, table.at[pl.ds(row0 + i * 64, 64)])
            return carry

        lax.fori_loop(0, rows_per_tile // 64, zstep, 0)
        plsc.subcore_barrier()

        def pair(j2, carry):
            for b in (0, 1):
                j = 2 * j2 + b
                k = k_of(j)

                @pl.when(k < NCH4)
                def _():
                    load_wait(b)
                load(j + 1, 1 - b)

                @pl.when(k < NCH4)
                def _():
                    for jj in range(2):
                        pltpu.sync_copy(
                            data_v.at[b, pl.ds(jj * 128, 128)],
                            table.at[idx_v.at[b, jj]], add=True)
            return carry

        lax.fori_loop(0, (NCH4 // 16 + 2) // 2, pair, 0)
        plsc.subcore_barrier()
        pltpu.sync_copy(
            table.at[pl.ds(row0, rows_per_tile)],
            out_hbm.at[c, pl.ds(row0, rows_per_tile), pl.ds(grp * 32, 32)])
        plsc.subcore_barrier()


def _k4(zy, dst2d):
    f = pl.kernel(
        _k4_body,
        out_type=jax.ShapeDtypeStruct((2, N_PAD, H), F32),
        mesh=_sc_mesh(),
        compiler_params=pltpu.CompilerParams(use_tc_tiling_on_sc=False),
        cost_estimate=pl.CostEstimate(
            flops=0, transcendentals=0, bytes_accessed=E * H * 4 * 2),
        scratch_types=[
            pltpu.VMEM((2, 2, 128), jnp.int32),
            pltpu.VMEM((2, 256, 32), F32),
            pltpu.VMEM((64, 32), F32),
            pltpu.VMEM_SHARED((N_PAD, 32), F32),
            pltpu.SemaphoreType.DMA,
        ],
    )
    return f(zy, dst2d)


def _k5_body(nn1, agz, agy, wa, wb, bun, w2, b2, wc, wd, bun2, wg, bg,
             t_o, sq_o):
    i = pl.program_id(0)
    nn = _relu(_dot(nn1[...], wa[...]) + _dot(agz[0], wb[...]) + bun[...])
    nn2 = _relu(_dot(nn, w2[...]) + b2[...])
    h = _relu(_dot(nn2, wc[...]) + _dot(agy[0], wd[...]) + bun2[...])
    t = jnp.tanh(_dot(h, wg[...]) + bg[...])
    rows = i * 512 + lax.broadcasted_iota(jnp.int32, (512, 1), 0)
    mask = (rows < N).astype(F32)
    t = t * mask
    t_o[...] = t
    sq_o[...] = jnp.sum(t * t, axis=0, keepdims=True)[None]


def _k5(nn1, aggs, wa, wb, bun, w2, b2, wc, wd, bun2, wg, bg):
    bs = 512
    row = pl.BlockSpec((bs, H), lambda i: (i, 0))
    agg0 = pl.BlockSpec((1, bs, H), lambda i: (0, i, 0))
    agg1 = pl.BlockSpec((1, bs, H), lambda i: (1, i, 0))
    wHH = pl.BlockSpec((H, H), lambda i: (0, 0))
    bia = pl.BlockSpec((1, H), lambda i: (0, 0))
    return pl.pallas_call(
        _k5_body,
        grid=(N_PAD // bs,),
        in_specs=[row, agg0, agg1, wHH, bia, wHH, bia, wHH, bia, wHH, bia][:3]
        + [wHH, wHH, bia, wHH, bia, wHH, wHH, bia, wHH, bia],
        out_specs=[
            pl.BlockSpec((bs, H), lambda i: (i, 0)),
            pl.BlockSpec((1, 1, H), lambda i: (i, 0, 0)),
        ],
        out_shape=[
            jax.ShapeDtypeStruct((N_PAD, H), F32),
            jax.ShapeDtypeStruct((NB5, 1, H), F32),
        ],
    )(nn1, aggs, aggs, wa, wb, bun, w2, b2, wc, wd, bun2, wg, bg)


def _k6_body(t_hbm, g2_hbm, gt_hbm, T_o, C_o, idx_v, dat_v, ones_v, zero_v,
             gidt_v, tabT, tabC):
    c = lax.axis_index("c")
    s = lax.axis_index("s")
    zf = jnp.zeros((16,), F32)
    of = jnp.full((16,), 1.0, F32)
    for r in range(16):
        for q in range(8):
            zero_v[r, pl.ds(q * 16, 16)] = zf
    for r in range(128):
        for q in range(8):
            ones_v[r, pl.ds(q * 16, 16)] = of
    for i in range(4):
        pltpu.sync_copy(zero_v, tabT.at[pl.ds(s * 64 + i * 16, 16)])
        pltpu.sync_copy(zero_v, tabC.at[pl.ds(s * 64 + i * 16, 16)])
    plsc.subcore_barrier()
    w = s * 2 + c

    def jstep(j, carry):
        k = w + 32 * j

        @pl.when(k < N_PAD // 128)
        def _():
            base = k * 128
            pltpu.sync_copy(g2_hbm.at[pl.ds(k, 1)], idx_v)
            pltpu.sync_copy(t_hbm.at[pl.ds(base, 128)], dat_v)
            pltpu.sync_copy(dat_v, tabT.at[idx_v.at[0]], add=True)

            @pl.when(k < N // 128)
            def _():
                pltpu.sync_copy(ones_v, tabC.at[idx_v.at[0]], add=True)
        return carry

    lax.fori_loop(0, 13, jstep, 0)

    @pl.when(w == 6)
    def _():
        pltpu.sync_copy(gt_hbm, gidt_v)
        pltpu.sync_copy(ones_v.at[pl.ds(0, N - (N // 128) * 128)],
                        tabC.at[gidt_v], add=True)

    plsc.subcore_barrier()
    pltpu.sync_copy(tabT.at[pl.ds(s * 64, 64)], T_o.at[c, pl.ds(s * 64, 64)])
    pltpu.sync_copy(tabC.at[pl.ds(s * 64, 64)], C_o.at[c, pl.ds(s * 64, 64)])


def _k6(t, gid2d, gid_tail):
    f = pl.kernel(
        _k6_body,
        out_type=[
            jax.ShapeDtypeStruct((2, G, H), F32),
            jax.ShapeDtypeStruct((2, G, H), F32),
        ],
        mesh=_sc_mesh(),
        compiler_params=pltpu.CompilerParams(use_tc_tiling_on_sc=False),
        scratch_types=[
            pltpu.VMEM((1, 128), jnp.int32),
            pltpu.VMEM((128, H), F32),
            pltpu.VMEM((128, H), F32),
            pltpu.VMEM((16, H), F32),
            pltpu.VMEM((N - (N // 128) * 128,), jnp.int32),
            pltpu.VMEM_SHARED((G, H), F32),
            pltpu.VMEM_SHARED((G, H), F32),
        ],
    )
    return f(t, gid2d, gid_tail)


def _k7_body(T1, C1, sq1, T2, C2, sq2, g1, b1, wpred, bpred, wfc, bfc,
             g2, b2, wout, bout, out_o):
    def gf(T, C, sq):
        Tt = T[0] + T[1]
        cnt = C[0, :, 0:1] + C[1, :, 0:1]
        s1 = jnp.sum(Tt, axis=0)
        s2 = jnp.sum(sq[...], axis=(0, 1))
        mu = s1 / N
        var = s2 / N - mu * mu
        s = g1[0] / jnp.sqrt(var + 1e-5)
        gfeat = s * Tt + cnt * (b1[0] - s * mu)
        return _dot(gfeat, wpred[...]) + bpred[...]

    d = gf(T1, C1, sq1) - gf(T2, C2, sq2)
    x = _dot(d, wfc[...]) + bfc[...]
    mu = jnp.mean(x, axis=0)
    var = jnp.mean((x - mu) ** 2, axis=0)
    x = _relu(g2[0] * (x - mu) / jnp.sqrt(var + 1e-5) + b2[0])
    out_o[...] = _dot(x, wout[...]) + bout[...]


def _k7(T1, C1, sq1, T2, C2, sq2, g1, b1, wpred, bpred, wfc, bfc, g2, b2,
        wout, bout):
    full = lambda shape: pl.BlockSpec(shape, lambda: tuple(0 for _ in shape))
    return pl.pallas_call(
        _k7_body,
        in_specs=[
            full((2, G, H)), full((2, G, H)), full((NB5, 1, H)),
            full((2, G, H)), full((2, G, H)), full((NB5, 1, H)),
            full((1, H)), full((1, H)),
            full((H, 256)), full((1, 256)),
            full((256, 512)), full((1, 512)),
            full((1, 512)), full((1, 512)),
            full((512, 1)), full((1, 1)),
        ],
        out_specs=full((G, 1)),
        out_shape=jax.ShapeDtypeStruct((G, 1), F32),
    )(T1, C1, sq1, T2, C2, sq2, g1, b1, wpred, bpred, wfc, bfc, g2, b2,
      wout, bout)


def kernel(node_feats1, edge_feats1, node_feats2, edge_feats2, edge_index1,
           graph_ids1, edge_index2, graph_ids2, params):
    p = params
    r2 = lambda v: v.reshape(1, -1)
    Wu_e, bu_e = p['l1_upd_e']
    Wu1, Wu2, Wu3 = Wu_e[:H], Wu_e[H:2 * H], Wu_e[2 * H:]
    Wu_n, bu_n = p['l1_upd_n']
    Wa, Wb = Wu_n[:H], Wu_n[H:]
    Wu_n2, bu_n2 = p['l2_upd_n']
    Wc, Wd = Wu_n2[:H], Wu_n2[H:]
    wn, bn = p['l1_n2n']
    wl, bl_ = p['l1_left']
    wr, br_ = p['l1_right']
    wzn, bzn = p['l1_e2n']
    wee, bee = p['l1_e2e']
    w4, b4 = p['l2_e2n']
    w2, b2 = p['l2_n2n']
    wg, bg = p['n2g']
    g1, b1 = p['bn1']
    wpred, bpred = p['pred']
    wfc, bfc = p['fc']
    g2, b2h = p['bn2']
    wout, bout = p['out']

    def branch(nf, ef, ei, gid):
        nfp = jnp.pad(nf, ((0, N_PAD - N), (0, 0)))
        src = ei[0]
        dst = ei[1]
        dst2d = dst.reshape(E // 128, 128)
        gid2d = jnp.pad(gid, (0, N_PAD - N)).reshape(N_PAD // 128, 128)
        gid_tail = gid[(N // 128) * 128:]
        nn1, A, B = _k1(nfp, wn, r2(bn), wl, r2(bl_), wr, r2(br_), Wu1, Wu2)
        pre = _k2(A, B, src, dst)
        zy = _k3(ef, pre, wzn, r2(bzn), wee, r2(bee), Wu3, r2(bu_e), w4,
                 r2(b4))
        aggs = _k4(zy, dst2d)
        t, sq = _k5(nn1, aggs, Wa, Wb, r2(bu_n), w2, r2(b2), Wc, Wd,
                    r2(bu_n2), wg, r2(bg))
        Tt, Ct = _k6(t, gid2d, gid_tail)
        return Tt, Ct, sq

    T1, C1, sq1 = branch(node_feats1, edge_feats1, edge_index1, graph_ids1)
    T2, C2, sq2 = branch(node_feats2, edge_feats2, edge_index2, graph_ids2)
    out = _k7(T1, C1, sq1, T2, C2, sq2, r2(g1), r2(b1), wpred, r2(bpred),
              wfc, r2(bfc), r2(g2), r2(b2h), wout, r2(bout))
    return out.reshape(G)


# speedup vs baseline: 1.1097x; 1.1097x over previous
import functools

import jax
import jax.numpy as jnp
from jax import lax
from jax.experimental import pallas as pl
from jax.experimental.pallas import tpu as pltpu
from jax.experimental.pallas import tpu_sc as plsc

N = 50000
E = 800000
G = 1024
H = 128
N_PAD = 50176
NB5 = N_PAD // 512
NCH = E // 1280
F32 = jnp.float32

@functools.lru_cache(maxsize=1)
def _sc_mesh():
    return plsc.VectorSubcoreMesh(
        core_axis_name="c", subcore_axis_name="s", num_cores=2,
        num_subcores=16)


def _relu(x):
    return jnp.maximum(x, 0.0)


def _dot(a, b):
    return jnp.dot(a, b, preferred_element_type=F32)


def _k1_body(nf, wn, bn, wl, bl_, wr, br_, wu1, wu2, nn1_o, a_o, b_o):
    x = nf[...]
    nn1_o[...] = _relu(_dot(x, wn[...]) + bn[...])
    a_o[...] = _dot(_relu(_dot(x, wl[...]) + bl_[...]), wu1[...])
    b_o[...] = _dot(_relu(_dot(x, wr[...]) + br_[...]), wu2[...])


def _k1(nfp, wn, bn, wl, bl_, wr, br_, wu1, wu2):
    bs = 512
    row = pl.BlockSpec((bs, 32), lambda i: (i, 0))
    w32 = pl.BlockSpec((32, H), lambda i: (0, 0))
    wHH = pl.BlockSpec((H, H), lambda i: (0, 0))
    bia = pl.BlockSpec((1, H), lambda i: (0, 0))
    out = pl.BlockSpec((bs, H), lambda i: (i, 0))
    return pl.pallas_call(
        _k1_body,
        grid=(N_PAD // bs,),
        in_specs=[row, w32, bia, w32, bia, w32, bia, wHH, wHH],
        out_specs=[out, out, out],
        out_shape=[jax.ShapeDtypeStruct((N_PAD, H), F32)] * 3,
    )(nfp, wn, bn, wl, bl_, wr, br_, wu1, wu2)


NCH2 = E // 128


def _k2_body(a_hbm, b_hbm, src_hbm, dst_hbm, out_hbm, si, di, buf,
             semi, semA, semB, semS):
    c = lax.axis_index("c")
    s = lax.axis_index("s")
    w = s * 2 + c

    def k_of(j):
        return w + 32 * j

    def idx_load(j, b):
        k = k_of(j)

        @pl.when(k < NCH2)
        def _():
            pltpu.async_copy(src_hbm.at[pl.ds(k * 128, 128)], si.at[b], semi)
            pltpu.async_copy(dst_hbm.at[pl.ds(k * 128, 128)], di.at[b], semi)

    def idx_wait(b):
        pltpu.make_async_copy(src_hbm.at[pl.ds(0, 128)], si.at[b], semi).wait()
        pltpu.make_async_copy(dst_hbm.at[pl.ds(0, 128)], di.at[b], semi).wait()

    idx_load(0, 0)
    idx_load(1, 1)
    idx_wait(0)
    pltpu.async_copy(a_hbm.at[si.at[0]], buf.at[0], semA)

    def pair(j2, carry):
        for b in (0, 1):
            j = 2 * j2 + b
            k = k_of(j)
            kn = k_of(j + 1)
            bn = 1 - b

            @pl.when(k < NCH2)
            def _():
                pltpu.make_async_copy(a_hbm.at[si.at[b]], buf.at[b],
                                      semA).wait()
                pltpu.async_copy(b_hbm.at[di.at[b]], buf.at[b], semB,
                                 add=True)

            @pl.when(kn < NCH2)
            def _():
                @pl.when(j >= 1)
                def _():
                    pltpu.make_async_copy(
                        buf.at[bn], out_hbm.at[pl.ds(0, 128)], semS).wait()
                idx_wait(bn)
                pltpu.async_copy(a_hbm.at[si.at[bn]], buf.at[bn], semA)

            @pl.when(k < NCH2)
            def _():
                pltpu.make_async_copy(b_hbm.at[di.at[b]], buf.at[b],
                                      semB).wait()
                pltpu.async_copy(buf.at[b], out_hbm.at[pl.ds(k * 128, 128)],
                                 semS)
            idx_load(j + 2, b)
        return carry

    lax.fori_loop(0, (NCH2 // 32 + 2) // 2, pair, 0)
    pltpu.make_async_copy(buf.at[0], out_hbm.at[pl.ds(0, 128)], semS).wait()
    pltpu.make_async_copy(buf.at[1], out_hbm.at[pl.ds(0, 128)], semS).wait()


def _k2(a_t, b_t, src, dst):
    f = pl.kernel(
        _k2_body,
        out_type=jax.ShapeDtypeStruct((E, H), F32),
        mesh=_sc_mesh(),
        compiler_params=pltpu.CompilerParams(use_tc_tiling_on_sc=False),
        cost_estimate=pl.CostEstimate(
            flops=0, transcendentals=0, bytes_accessed=E * H * 4 * 3),
        scratch_types=[
            pltpu.VMEM((2, 128), jnp.int32),
            pltpu.VMEM((2, 128), jnp.int32),
            pltpu.VMEM((2, 128, H), F32),
            pltpu.SemaphoreType.DMA,
            pltpu.SemaphoreType.DMA,
            pltpu.SemaphoreType.DMA,
            pltpu.SemaphoreType.DMA,
        ],
    )
    return f(a_t, b_t, src, dst)


def _k3_body(eft, pre, wzn, bzn, wee, bee, wu3, bue, w4, b4, zy_o):
    bf = jnp.bfloat16
    et = eft[...].astype(bf)

    def tdot(a, b):
        return lax.dot_general(a, b, (((0,), (0,)), ((), ())),
                               preferred_element_type=F32)

    z = _relu(tdot(et, wzn[...].astype(bf)) + bzn[...])
    e2e = _relu(tdot(et, wee[...].astype(bf)) + bee[...])
    cc = _dot(e2e.astype(bf), wu3[...].astype(bf)) + bue[...]
    ne = _relu(pre[...] + cc)
    y = _relu(_dot(ne.astype(bf), w4[...].astype(bf)) + b4[...])
    zy_o[0, :, :] = z
    zy_o[1, :, :] = y


def _k3(eft, pre, wzn, bzn, wee, bee, wu3, bue, w4, b4):
    bs = 640
    return pl.pallas_call(
        _k3_body,
        grid=(E // bs,),
        in_specs=[
            pl.BlockSpec((6, bs), lambda i: (0, i)),
            pl.BlockSpec((bs, H), lambda i: (i, 0)),
            pl.BlockSpec((6, H), lambda i: (0, 0)),
            pl.BlockSpec((1, H), lambda i: (0, 0)),
            pl.BlockSpec((6, H), lambda i: (0, 0)),
            pl.BlockSpec((1, H), lambda i: (0, 0)),
            pl.BlockSpec((H, H), lambda i: (0, 0)),
            pl.BlockSpec((1, H), lambda i: (0, 0)),
            pl.BlockSpec((H, H), lambda i: (0, 0)),
            pl.BlockSpec((1, H), lambda i: (0, 0)),
        ],
        out_specs=pl.BlockSpec((2, bs, H), lambda i: (0, i, 0)),
        out_shape=jax.ShapeDtypeStruct((2, E, H), F32),
    )(eft, pre, wzn, bzn, wee, bee, wu3, bue, w4, b4)


NCH4 = E // 256


def _k4_body(zy_hbm, d2_hbm, out_hbm, idx_v, data_v, zero_v, table, semL):
    c = lax.axis_index("c")
    s = lax.axis_index("s")
    zf = jnp.zeros((16,), F32)
    for r in range(64):
        for q in range(2):
            zero_v[r, pl.ds(q * 16, 16)] = zf

    rows_per_tile = N_PAD // 16
    row0 = s * rows_per_tile

    def k_of(j):
        return s + 16 * j

    for grp in range(4):
        def load(j, b):
            k = k_of(j)

            @pl.when(k < NCH4)
            def _():
                pltpu.async_copy(d2_hbm.at[pl.ds(k * 2, 2)], idx_v.at[b],
                                 semL)
                pltpu.async_copy(
                    zy_hbm.at[c, pl.ds(k * 256, 256), pl.ds(grp * 32, 32)],
                    data_v.at[b], semL)

        def load_wait(b):
            pltpu.make_async_copy(d2_hbm.at[pl.ds(0, 2)], idx_v.at[b],
                                  semL).wait()
            pltpu.make_async_copy(
                zy_hbm.at[c, pl.ds(0, 256), pl.ds(grp * 32, 32)],
                data_v.at[b], semL).wait()

        load(0, 0)

        def zstep(i, carry):
            pltpu.sync_copy(zero_v, table.at[pl.ds(row0 + i * 64, 64)])
            return carry

        lax.fori_loop(0, rows_per_tile // 64, zstep, 0)
        plsc.subcore_barrier()

        def pair(j2, carry):
            for b in (0, 1):
                j = 2 * j2 + b
                k = k_of(j)

                @pl.when(k < NCH4)
                def _():
                    load_wait(b)
                load(j + 1, 1 - b)

                @pl.when(k < NCH4)
                def _():
                    for jj in range(2):
                        pltpu.sync_copy(
                            data_v.at[b, pl.ds(jj * 128, 128)],
                            table.at[idx_v.at[b, jj]], add=True)
            return carry

        lax.fori_loop(0, (NCH4 // 16 + 2) // 2, pair, 0)
        plsc.subcore_barrier()
        pltpu.sync_copy(
            table.at[pl.ds(row0, rows_per_tile)],
            out_hbm.at[c, pl.ds(row0, rows_per_tile), pl.ds(grp * 32, 32)])
        plsc.subcore_barrier()


def _k4(zy, dst2d):
    f = pl.kernel(
        _k4_body,
        out_type=jax.ShapeDtypeStruct((2, N_PAD, H), F32),
        mesh=_sc_mesh(),
        compiler_params=pltpu.CompilerParams(use_tc_tiling_on_sc=False),
        cost_estimate=pl.CostEstimate(
            flops=0, transcendentals=0, bytes_accessed=E * H * 4 * 2),
        scratch_types=[
            pltpu.VMEM((2, 2, 128), jnp.int32),
            pltpu.VMEM((2, 256, 32), F32),
            pltpu.VMEM((64, 32), F32),
            pltpu.VMEM_SHARED((N_PAD, 32), F32),
            pltpu.SemaphoreType.DMA,
        ],
    )
    return f(zy, dst2d)


def _k5_body(nn1, agz, agy, wa, wb, bun, w2, b2, wc, wd, bun2, wg, bg,
             t_o, sq_o):
    i = pl.program_id(0)
    nn = _relu(_dot(nn1[...], wa[...]) + _dot(agz[0], wb[...]) + bun[...])
    nn2 = _relu(_dot(nn, w2[...]) + b2[...])
    h = _relu(_dot(nn2, wc[...]) + _dot(agy[0], wd[...]) + bun2[...])
    t = jnp.tanh(_dot(h, wg[...]) + bg[...])
    rows = i * 512 + lax.broadcasted_iota(jnp.int32, (512, 1), 0)
    mask = (rows < N).astype(F32)
    t = t * mask
    t_o[...] = t
    sq_o[...] = jnp.sum(t * t, axis=0, keepdims=True)[None]


def _k5(nn1, aggs, wa, wb, bun, w2, b2, wc, wd, bun2, wg, bg):
    bs = 512
    row = pl.BlockSpec((bs, H), lambda i: (i, 0))
    agg0 = pl.BlockSpec((1, bs, H), lambda i: (0, i, 0))
    agg1 = pl.BlockSpec((1, bs, H), lambda i: (1, i, 0))
    wHH = pl.BlockSpec((H, H), lambda i: (0, 0))
    bia = pl.BlockSpec((1, H), lambda i: (0, 0))
    return pl.pallas_call(
        _k5_body,
        grid=(N_PAD // bs,),
        in_specs=[row, agg0, agg1, wHH, bia, wHH, bia, wHH, bia, wHH, bia][:3]
        + [wHH, wHH, bia, wHH, bia, wHH, wHH, bia, wHH, bia],
        out_specs=[
            pl.BlockSpec((bs, H), lambda i: (i, 0)),
            pl.BlockSpec((1, 1, H), lambda i: (i, 0, 0)),
        ],
        out_shape=[
            jax.ShapeDtypeStruct((N_PAD, H), F32),
            jax.ShapeDtypeStruct((NB5, 1, H), F32),
        ],
    )(nn1, aggs, aggs, wa, wb, bun, w2, b2, wc, wd, bun2, wg, bg)


def _k6_body(t_hbm, g2_hbm, gt_hbm, T_o, C_o, idx_v, dat_v, ones_v, zero_v,
             gidt_v, tabT, tabC):
    c = lax.axis_index("c")
    s = lax.axis_index("s")
    zf = jnp.zeros((16,), F32)
    of = jnp.full((16,), 1.0, F32)
    for r in range(16):
        for q in range(8):
            zero_v[r, pl.ds(q * 16, 16)] = zf
    for r in range(128):
        for q in range(8):
            ones_v[r, pl.ds(q * 16, 16)] = of
    for i in range(4):
        pltpu.sync_copy(zero_v, tabT.at[pl.ds(s * 64 + i * 16, 16)])
        pltpu.sync_copy(zero_v, tabC.at[pl.ds(s * 64 + i * 16, 16)])
    plsc.subcore_barrier()
    w = s * 2 + c

    def jstep(j, carry):
        k = w + 32 * j

        @pl.when(k < N_PAD // 128)
        def _():
            base = k * 128
            pltpu.sync_copy(g2_hbm.at[pl.ds(k, 1)], idx_v)
            pltpu.sync_copy(t_hbm.at[pl.ds(base, 128)], dat_v)
            pltpu.sync_copy(dat_v, tabT.at[idx_v.at[0]], add=True)

            @pl.when(k < N // 128)
            def _():
                pltpu.sync_copy(ones_v, tabC.at[idx_v.at[0]], add=True)
        return carry

    lax.fori_loop(0, 13, jstep, 0)

    @pl.when(w == 6)
    def _():
        pltpu.sync_copy(gt_hbm, gidt_v)
        pltpu.sync_copy(ones_v.at[pl.ds(0, N - (N // 128) * 128)],
                        tabC.at[gidt_v], add=True)

    plsc.subcore_barrier()
    pltpu.sync_copy(tabT.at[pl.ds(s * 64, 64)], T_o.at[c, pl.ds(s * 64, 64)])
    pltpu.sync_copy(tabC.at[pl.ds(s * 64, 64)], C_o.at[c, pl.ds(s * 64, 64)])


def _k6(t, gid2d, gid_tail):
    f = pl.kernel(
        _k6_body,
        out_type=[
            jax.ShapeDtypeStruct((2, G, H), F32),
            jax.ShapeDtypeStruct((2, G, H), F32),
        ],
        mesh=_sc_mesh(),
        compiler_params=pltpu.CompilerParams(use_tc_tiling_on_sc=False),
        scratch_types=[
            pltpu.VMEM((1, 128), jnp.int32),
            pltpu.VMEM((128, H), F32),
            pltpu.VMEM((128, H), F32),
            pltpu.VMEM((16, H), F32),
            pltpu.VMEM((N - (N // 128) * 128,), jnp.int32),
            pltpu.VMEM_SHARED((G, H), F32),
            pltpu.VMEM_SHARED((G, H), F32),
        ],
    )
    return f(t, gid2d, gid_tail)


def _k7_body(T1, C1, sq1, T2, C2, sq2, g1, b1, wpred, bpred, wfc, bfc,
             g2, b2, wout, bout, out_o):
    def gf(T, C, sq):
        Tt = T[0] + T[1]
        cnt = C[0, :, 0:1] + C[1, :, 0:1]
        s1 = jnp.sum(Tt, axis=0)
        s2 = jnp.sum(sq[...], axis=(0, 1))
        mu = s1 / N
        var = s2 / N - mu * mu
        s = g1[0] / jnp.sqrt(var + 1e-5)
        gfeat = s * Tt + cnt * (b1[0] - s * mu)
        return _dot(gfeat, wpred[...]) + bpred[...]

    d = gf(T1, C1, sq1) - gf(T2, C2, sq2)
    x = _dot(d, wfc[...]) + bfc[...]
    mu = jnp.mean(x, axis=0)
    var = jnp.mean((x - mu) ** 2, axis=0)
    x = _relu(g2[0] * (x - mu) / jnp.sqrt(var + 1e-5) + b2[0])
    out_o[...] = _dot(x, wout[...]) + bout[...]


def _k7(T1, C1, sq1, T2, C2, sq2, g1, b1, wpred, bpred, wfc, bfc, g2, b2,
        wout, bout):
    full = lambda shape: pl.BlockSpec(shape, lambda: tuple(0 for _ in shape))
    return pl.pallas_call(
        _k7_body,
        in_specs=[
            full((2, G, H)), full((2, G, H)), full((NB5, 1, H)),
            full((2, G, H)), full((2, G, H)), full((NB5, 1, H)),
            full((1, H)), full((1, H)),
            full((H, 256)), full((1, 256)),
            full((256, 512)), full((1, 512)),
            full((1, 512)), full((1, 512)),
            full((512, 1)), full((1, 1)),
        ],
        out_specs=full((G, 1)),
        out_shape=jax.ShapeDtypeStruct((G, 1), F32),
    )(T1, C1, sq1, T2, C2, sq2, g1, b1, wpred, bpred, wfc, bfc, g2, b2,
      wout, bout)


def kernel(node_feats1, edge_feats1, node_feats2, edge_feats2, edge_index1,
           graph_ids1, edge_index2, graph_ids2, params):
    p = params
    r2 = lambda v: v.reshape(1, -1)
    Wu_e, bu_e = p['l1_upd_e']
    Wu1, Wu2, Wu3 = Wu_e[:H], Wu_e[H:2 * H], Wu_e[2 * H:]
    Wu_n, bu_n = p['l1_upd_n']
    Wa, Wb = Wu_n[:H], Wu_n[H:]
    Wu_n2, bu_n2 = p['l2_upd_n']
    Wc, Wd = Wu_n2[:H], Wu_n2[H:]
    wn, bn = p['l1_n2n']
    wl, bl_ = p['l1_left']
    wr, br_ = p['l1_right']
    wzn, bzn = p['l1_e2n']
    wee, bee = p['l1_e2e']
    w4, b4 = p['l2_e2n']
    w2, b2 = p['l2_n2n']
    wg, bg = p['n2g']
    g1, b1 = p['bn1']
    wpred, bpred = p['pred']
    wfc, bfc = p['fc']
    g2, b2h = p['bn2']
    wout, bout = p['out']

    def branch(nf, ef, ei, gid):
        nfp = jnp.pad(nf, ((0, N_PAD - N), (0, 0)))
        src = ei[0]
        dst = ei[1]
        dst2d = dst.reshape(E // 128, 128)
        gid2d = jnp.pad(gid, (0, N_PAD - N)).reshape(N_PAD // 128, 128)
        gid_tail = gid[(N // 128) * 128:]
        nn1, A, B = _k1(nfp, wn, r2(bn), wl, r2(bl_), wr, r2(br_), Wu1, Wu2)
        pre = _k2(A, B, src, dst)
        zy = _k3(ef.T, pre, wzn, r2(bzn), wee, r2(bee), Wu3, r2(bu_e), w4,
                 r2(b4))
        aggs = _k4(zy, dst2d)
        t, sq = _k5(nn1, aggs, Wa, Wb, r2(bu_n), w2, r2(b2), Wc, Wd,
                    r2(bu_n2), wg, r2(bg))
        Tt, Ct = _k6(t, gid2d, gid_tail)
        return Tt, Ct, sq

    T1, C1, sq1 = branch(node_feats1, edge_feats1, edge_index1, graph_ids1)
    T2, C2, sq2 = branch(node_feats2, edge_feats2, edge_index2, graph_ids2)
    out = _k7(T1, C1, sq1, T2, C2, sq2, r2(g1), r2(b1), wpred, r2(bpred),
              wfc, r2(bfc), r2(g2), r2(b2h), wout, r2(bout))
    return out.reshape(G)


# speedup vs baseline: 1.2976x; 1.1694x over previous
import functools

import jax
import jax.numpy as jnp
from jax import lax
from jax.experimental import pallas as pl
from jax.experimental.pallas import tpu as pltpu
from jax.experimental.pallas import tpu_sc as plsc

N = 50000
E = 800000
G = 1024
H = 128
N_PAD = 50176
NB5 = N_PAD // 512
NCH = E // 1280
F32 = jnp.float32

@functools.lru_cache(maxsize=1)
def _sc_mesh():
    return plsc.VectorSubcoreMesh(
        core_axis_name="c", subcore_axis_name="s", num_cores=2,
        num_subcores=16)


def _relu(x):
    return jnp.maximum(x, 0.0)


def _dot(a, b):
    return jnp.dot(a, b, preferred_element_type=F32)


def _k1_body(nf, wn, bn, wl, bl_, wr, br_, wu1, wu2, nn1_o, a_o, b_o):
    x = nf[...]
    nn1_o[...] = _relu(_dot(x, wn[...]) + bn[...])
    a_o[...] = _dot(_relu(_dot(x, wl[...]) + bl_[...]), wu1[...])
    b_o[...] = _dot(_relu(_dot(x, wr[...]) + br_[...]), wu2[...])


def _k1(nfp, wn, bn, wl, bl_, wr, br_, wu1, wu2):
    bs = 512
    row = pl.BlockSpec((bs, 32), lambda i: (i, 0))
    w32 = pl.BlockSpec((32, H), lambda i: (0, 0))
    wHH = pl.BlockSpec((H, H), lambda i: (0, 0))
    bia = pl.BlockSpec((1, H), lambda i: (0, 0))
    out = pl.BlockSpec((bs, H), lambda i: (i, 0))
    return pl.pallas_call(
        _k1_body,
        grid=(N_PAD // bs,),
        in_specs=[row, w32, bia, w32, bia, w32, bia, wHH, wHH],
        out_specs=[out, out, out],
        out_shape=[jax.ShapeDtypeStruct((N_PAD, H), F32)] * 3,
    )(nfp, wn, bn, wl, bl_, wr, br_, wu1, wu2)


NCH2 = E // 128


def _k2_body(a_hbm, b_hbm, src_hbm, dst_hbm, out_hbm, si, di, buf,
             semi, semA, semB, semS):
    c = lax.axis_index("c")
    s = lax.axis_index("s")
    w = s * 2 + c

    def k_of(j):
        return w + 32 * j

    def idx_load(j, b):
        k = k_of(j)

        @pl.when(k < NCH2)
        def _():
            pltpu.async_copy(src_hbm.at[pl.ds(k * 128, 128)], si.at[b], semi)
            pltpu.async_copy(dst_hbm.at[pl.ds(k * 128, 128)], di.at[b], semi)

    def idx_wait(b):
        pltpu.make_async_copy(src_hbm.at[pl.ds(0, 128)], si.at[b], semi).wait()
        pltpu.make_async_copy(dst_hbm.at[pl.ds(0, 128)], di.at[b], semi).wait()

    idx_load(0, 0)
    idx_load(1, 1)
    idx_wait(0)
    pltpu.async_copy(a_hbm.at[si.at[0]], buf.at[0], semA)

    def pair(j2, carry):
        for b in (0, 1):
            j = 2 * j2 + b
            k = k_of(j)
            kn = k_of(j + 1)
            bn = 1 - b

            @pl.when(k < NCH2)
            def _():
                pltpu.make_async_copy(a_hbm.at[si.at[b]], buf.at[b],
                                      semA).wait()
                pltpu.async_copy(b_hbm.at[di.at[b]], buf.at[b], semB,
                                 add=True)

            @pl.when(kn < NCH2)
            def _():
                @pl.when(j >= 1)
                def _():
                    pltpu.make_async_copy(
                        buf.at[bn], out_hbm.at[pl.ds(0, 128)], semS).wait()
                idx_wait(bn)
                pltpu.async_copy(a_hbm.at[si.at[bn]], buf.at[bn], semA)

            @pl.when(k < NCH2)
            def _():
                pltpu.make_async_copy(b_hbm.at[di.at[b]], buf.at[b],
                                      semB).wait()
                pltpu.async_copy(buf.at[b], out_hbm.at[pl.ds(k * 128, 128)],
                                 semS)
            idx_load(j + 2, b)
        return carry

    lax.fori_loop(0, (NCH2 // 32 + 2) // 2, pair, 0)
    pltpu.make_async_copy(buf.at[0], out_hbm.at[pl.ds(0, 128)], semS).wait()
    pltpu.make_async_copy(buf.at[1], out_hbm.at[pl.ds(0, 128)], semS).wait()


def _k2(a_t, b_t, src, dst):
    f = pl.kernel(
        _k2_body,
        out_type=jax.ShapeDtypeStruct((E, H), F32),
        mesh=_sc_mesh(),
        compiler_params=pltpu.CompilerParams(use_tc_tiling_on_sc=False),
        cost_estimate=pl.CostEstimate(
            flops=0, transcendentals=0, bytes_accessed=E * H * 4 * 3),
        scratch_types=[
            pltpu.VMEM((2, 128), jnp.int32),
            pltpu.VMEM((2, 128), jnp.int32),
            pltpu.VMEM((2, 128, H), F32),
            pltpu.SemaphoreType.DMA,
            pltpu.SemaphoreType.DMA,
            pltpu.SemaphoreType.DMA,
            pltpu.SemaphoreType.DMA,
        ],
    )
    return f(a_t, b_t, src, dst)


def _k3_body(eft, pre, wzn, bzn, wee, bee, wu3, bue, w4, b4, zy_o):
    bf = jnp.bfloat16
    et = eft[...].astype(bf)

    def tdot(a, b):
        return lax.dot_general(a, b, (((0,), (0,)), ((), ())),
                               preferred_element_type=F32)

    z = _relu(tdot(et, wzn[...].astype(bf)) + bzn[...])
    e2e = _relu(tdot(et, wee[...].astype(bf)) + bee[...])
    cc = _dot(e2e.astype(bf), wu3[...].astype(bf)) + bue[...]
    ne = _relu(pre[...] + cc)
    y = _relu(_dot(ne.astype(bf), w4[...].astype(bf)) + b4[...])
    zy_o[0, :, :] = z
    zy_o[1, :, :] = y


def _k3(eft, pre, wzn, bzn, wee, bee, wu3, bue, w4, b4):
    bs = 3200
    return pl.pallas_call(
        _k3_body,
        grid=(E // bs,),
        in_specs=[
            pl.BlockSpec((6, bs), lambda i: (0, i)),
            pl.BlockSpec((bs, H), lambda i: (i, 0)),
            pl.BlockSpec((6, H), lambda i: (0, 0)),
            pl.BlockSpec((1, H), lambda i: (0, 0)),
            pl.BlockSpec((6, H), lambda i: (0, 0)),
            pl.BlockSpec((1, H), lambda i: (0, 0)),
            pl.BlockSpec((H, H), lambda i: (0, 0)),
            pl.BlockSpec((1, H), lambda i: (0, 0)),
            pl.BlockSpec((H, H), lambda i: (0, 0)),
            pl.BlockSpec((1, H), lambda i: (0, 0)),
        ],
        out_specs=pl.BlockSpec((2, bs, H), lambda i: (0, i, 0)),
        out_shape=jax.ShapeDtypeStruct((2, E, H), F32),
    )(eft, pre, wzn, bzn, wee, bee, wu3, bue, w4, b4)


NCH4 = E // 256


def _k4_body(zy_hbm, d2_hbm, out_hbm, idx_v, data_v, zero_v, table, semL):
    c = lax.axis_index("c")
    s = lax.axis_index("s")
    zf = jnp.zeros((16,), F32)
    for r in range(64):
        for q in range(2):
            zero_v[r, pl.ds(q * 16, 16)] = zf

    rows_per_tile = N_PAD // 16
    row0 = s * rows_per_tile

    def k_of(j):
        return s + 16 * j

    for grp in range(4):
        def load(j, b):
            k = k_of(j)

            @pl.when(k < NCH4)
            def _():
                pltpu.async_copy(d2_hbm.at[pl.ds(k * 2, 2)], idx_v.at[b],
                                 semL)
                pltpu.async_copy(
                    zy_hbm.at[c, pl.ds(k * 256, 256), pl.ds(grp * 32, 32)],
                    data_v.at[b], semL)

        def load_wait(b):
            pltpu.make_async_copy(d2_hbm.at[pl.ds(0, 2)], idx_v.at[b],
                                  semL).wait()
            pltpu.make_async_copy(
                zy_hbm.at[c, pl.ds(0, 256), pl.ds(grp * 32, 32)],
                data_v.at[b], semL).wait()

        load(0, 0)

        def zstep(i, carry):
            pltpu.sync_copy(zero_v, table.at[pl.ds(row0 + i * 64, 64)])
            return carry

        lax.fori_loop(0, rows_per_tile // 64, zstep, 0)
        plsc.subcore_barrier()

        def pair(j2, carry):
            for b in (0, 1):
                j = 2 * j2 + b
                k = k_of(j)

                @pl.when(k < NCH4)
                def _():
                    load_wait(b)
                load(j + 1, 1 - b)

                @pl.when(k < NCH4)
                def _():
                    for jj in range(2):
                        pltpu.sync_copy(
                            data_v.at[b, pl.ds(jj * 128, 128)],
                            table.at[idx_v.at[b, jj]], add=True)
            return carry

        lax.fori_loop(0, (NCH4 // 16 + 2) // 2, pair, 0)
        plsc.subcore_barrier()
        pltpu.sync_copy(
            table.at[pl.ds(row0, rows_per_tile)],
            out_hbm.at[c, pl.ds(row0, rows_per_tile), pl.ds(grp * 32, 32)])
        plsc.subcore_barrier()


def _k4(zy, dst2d):
    f = pl.kernel(
        _k4_body,
        out_type=jax.ShapeDtypeStruct((2, N_PAD, H), F32),
        mesh=_sc_mesh(),
        compiler_params=pltpu.CompilerParams(use_tc_tiling_on_sc=False),
        cost_estimate=pl.CostEstimate(
            flops=0, transcendentals=0, bytes_accessed=E * H * 4 * 2),
        scratch_types=[
            pltpu.VMEM((2, 2, 128), jnp.int32),
            pltpu.VMEM((2, 256, 32), F32),
            pltpu.VMEM((64, 32), F32),
            pltpu.VMEM_SHARED((N_PAD, 32), F32),
            pltpu.SemaphoreType.DMA,
        ],
    )
    return f(zy, dst2d)


def _k5_body(nn1, agz, agy, wa, wb, bun, w2, b2, wc, wd, bun2, wg, bg,
             t_o, sq_o):
    i = pl.program_id(0)
    nn = _relu(_dot(nn1[...], wa[...]) + _dot(agz[0], wb[...]) + bun[...])
    nn2 = _relu(_dot(nn, w2[...]) + b2[...])
    h = _relu(_dot(nn2, wc[...]) + _dot(agy[0], wd[...]) + bun2[...])
    t = jnp.tanh(_dot(h, wg[...]) + bg[...])
    rows = i * 512 + lax.broadcasted_iota(jnp.int32, (512, 1), 0)
    mask = (rows < N).astype(F32)
    t = t * mask
    t_o[...] = t
    sq_o[...] = jnp.sum(t * t, axis=0, keepdims=True)[None]


def _k5(nn1, aggs, wa, wb, bun, w2, b2, wc, wd, bun2, wg, bg):
    bs = 512
    row = pl.BlockSpec((bs, H), lambda i: (i, 0))
    agg0 = pl.BlockSpec((1, bs, H), lambda i: (0, i, 0))
    agg1 = pl.BlockSpec((1, bs, H), lambda i: (1, i, 0))
    wHH = pl.BlockSpec((H, H), lambda i: (0, 0))
    bia = pl.BlockSpec((1, H), lambda i: (0, 0))
    return pl.pallas_call(
        _k5_body,
        grid=(N_PAD // bs,),
        in_specs=[row, agg0, agg1, wHH, bia, wHH, bia, wHH, bia, wHH, bia][:3]
        + [wHH, wHH, bia, wHH, bia, wHH, wHH, bia, wHH, bia],
        out_specs=[
            pl.BlockSpec((bs, H), lambda i: (i, 0)),
            pl.BlockSpec((1, 1, H), lambda i: (i, 0, 0)),
        ],
        out_shape=[
            jax.ShapeDtypeStruct((N_PAD, H), F32),
            jax.ShapeDtypeStruct((NB5, 1, H), F32),
        ],
    )(nn1, aggs, aggs, wa, wb, bun, w2, b2, wc, wd, bun2, wg, bg)


def _k6_body(t_hbm, g2_hbm, gt_hbm, T_o, C_o, idx_v, dat_v, ones_v, zero_v,
             gidt_v, tabT, tabC):
    c = lax.axis_index("c")
    s = lax.axis_index("s")
    zf = jnp.zeros((16,), F32)
    of = jnp.full((16,), 1.0, F32)
    for r in range(16):
        for q in range(8):
            zero_v[r, pl.ds(q * 16, 16)] = zf
    for r in range(128):
        for q in range(8):
            ones_v[r, pl.ds(q * 16, 16)] = of
    for i in range(4):
        pltpu.sync_copy(zero_v, tabT.at[pl.ds(s * 64 + i * 16, 16)])
        pltpu.sync_copy(zero_v, tabC.at[pl.ds(s * 64 + i * 16, 16)])
    plsc.subcore_barrier()
    w = s * 2 + c

    def jstep(j, carry):
        k = w + 32 * j

        @pl.when(k < N_PAD // 128)
        def _():
            base = k * 128
            pltpu.sync_copy(g2_hbm.at[pl.ds(k, 1)], idx_v)
            pltpu.sync_copy(t_hbm.at[pl.ds(base, 128)], dat_v)
            pltpu.sync_copy(dat_v, tabT.at[idx_v.at[0]], add=True)

            @pl.when(k < N // 128)
            def _():
                pltpu.sync_copy(ones_v, tabC.at[idx_v.at[0]], add=True)
        return carry

    lax.fori_loop(0, 13, jstep, 0)

    @pl.when(w == 6)
    def _():
        pltpu.sync_copy(gt_hbm, gidt_v)
        pltpu.sync_copy(ones_v.at[pl.ds(0, N - (N // 128) * 128)],
                        tabC.at[gidt_v], add=True)

    plsc.subcore_barrier()
    pltpu.sync_copy(tabT.at[pl.ds(s * 64, 64)], T_o.at[c, pl.ds(s * 64, 64)])
    pltpu.sync_copy(tabC.at[pl.ds(s * 64, 64)], C_o.at[c, pl.ds(s * 64, 64)])


def _k6(t, gid2d, gid_tail):
    f = pl.kernel(
        _k6_body,
        out_type=[
            jax.ShapeDtypeStruct((2, G, H), F32),
            jax.ShapeDtypeStruct((2, G, H), F32),
        ],
        mesh=_sc_mesh(),
        compiler_params=pltpu.CompilerParams(use_tc_tiling_on_sc=False),
        scratch_types=[
            pltpu.VMEM((1, 128), jnp.int32),
            pltpu.VMEM((128, H), F32),
            pltpu.VMEM((128, H), F32),
            pltpu.VMEM((16, H), F32),
            pltpu.VMEM((N - (N // 128) * 128,), jnp.int32),
            pltpu.VMEM_SHARED((G, H), F32),
            pltpu.VMEM_SHARED((G, H), F32),
        ],
    )
    return f(t, gid2d, gid_tail)


def _k7_body(T1, C1, sq1, T2, C2, sq2, g1, b1, wpred, bpred, wfc, bfc,
             g2, b2, wout, bout, out_o):
    def gf(T, C, sq):
        Tt = T[0] + T[1]
        cnt = C[0, :, 0:1] + C[1, :, 0:1]
        s1 = jnp.sum(Tt, axis=0)
        s2 = jnp.sum(sq[...], axis=(0, 1))
        mu = s1 / N
        var = s2 / N - mu * mu
        s = g1[0] / jnp.sqrt(var + 1e-5)
        gfeat = s * Tt + cnt * (b1[0] - s * mu)
        return _dot(gfeat, wpred[...]) + bpred[...]

    d = gf(T1, C1, sq1) - gf(T2, C2, sq2)
    x = _dot(d, wfc[...]) + bfc[...]
    mu = jnp.mean(x, axis=0)
    var = jnp.mean((x - mu) ** 2, axis=0)
    x = _relu(g2[0] * (x - mu) / jnp.sqrt(var + 1e-5) + b2[0])
    out_o[...] = _dot(x, wout[...]) + bout[...]


def _k7(T1, C1, sq1, T2, C2, sq2, g1, b1, wpred, bpred, wfc, bfc, g2, b2,
        wout, bout):
    full = lambda shape: pl.BlockSpec(shape, lambda: tuple(0 for _ in shape))
    return pl.pallas_call(
        _k7_body,
        in_specs=[
            full((2, G, H)), full((2, G, H)), full((NB5, 1, H)),
            full((2, G, H)), full((2, G, H)), full((NB5, 1, H)),
            full((1, H)), full((1, H)),
            full((H, 256)), full((1, 256)),
            full((256, 512)), full((1, 512)),
            full((1, 512)), full((1, 512)),
            full((512, 1)), full((1, 1)),
        ],
        out_specs=full((G, 1)),
        out_shape=jax.ShapeDtypeStruct((G, 1), F32),
    )(T1, C1, sq1, T2, C2, sq2, g1, b1, wpred, bpred, wfc, bfc, g2, b2,
      wout, bout)


def kernel(node_feats1, edge_feats1, node_feats2, edge_feats2, edge_index1,
           graph_ids1, edge_index2, graph_ids2, params):
    p = params
    r2 = lambda v: v.reshape(1, -1)
    Wu_e, bu_e = p['l1_upd_e']
    Wu1, Wu2, Wu3 = Wu_e[:H], Wu_e[H:2 * H], Wu_e[2 * H:]
    Wu_n, bu_n = p['l1_upd_n']
    Wa, Wb = Wu_n[:H], Wu_n[H:]
    Wu_n2, bu_n2 = p['l2_upd_n']
    Wc, Wd = Wu_n2[:H], Wu_n2[H:]
    wn, bn = p['l1_n2n']
    wl, bl_ = p['l1_left']
    wr, br_ = p['l1_right']
    wzn, bzn = p['l1_e2n']
    wee, bee = p['l1_e2e']
    w4, b4 = p['l2_e2n']
    w2, b2 = p['l2_n2n']
    wg, bg = p['n2g']
    g1, b1 = p['bn1']
    wpred, bpred = p['pred']
    wfc, bfc = p['fc']
    g2, b2h = p['bn2']
    wout, bout = p['out']

    def branch(nf, ef, ei, gid):
        nfp = jnp.pad(nf, ((0, N_PAD - N), (0, 0)))
        src = ei[0]
        dst = ei[1]
        dst2d = dst.reshape(E // 128, 128)
        gid2d = jnp.pad(gid, (0, N_PAD - N)).reshape(N_PAD // 128, 128)
        gid_tail = gid[(N // 128) * 128:]
        nn1, A, B = _k1(nfp, wn, r2(bn), wl, r2(bl_), wr, r2(br_), Wu1, Wu2)
        pre = _k2(A, B, src, dst)
        zy = _k3(ef.T, pre, wzn, r2(bzn), wee, r2(bee), Wu3, r2(bu_e), w4,
                 r2(b4))
        aggs = _k4(zy, dst2d)
        t, sq = _k5(nn1, aggs, Wa, Wb, r2(bu_n), w2, r2(b2), Wc, Wd,
                    r2(bu_n2), wg, r2(bg))
        Tt, Ct = _k6(t, gid2d, gid_tail)
        return Tt, Ct, sq

    T1, C1, sq1 = branch(node_feats1, edge_feats1, edge_index1, graph_ids1)
    T2, C2, sq2 = branch(node_feats2, edge_feats2, edge_index2, graph_ids2)
    out = _k7(T1, C1, sq1, T2, C2, sq2, r2(g1), r2(b1), wpred, r2(bpred),
              wfc, r2(bfc), r2(g2), r2(b2h), wout, r2(bout))
    return out.reshape(G)
